# Initial kernel scaffold; baseline (speedup 1.0000x reference)
import jax
import jax.numpy as jnp
from jax.experimental import pallas as pl


def kernel(adj):
    raise NotImplementedError("write your pallas kernel here")


# speedup vs baseline: 36.5656x; 36.5656x over previous
import functools

import jax
import jax.numpy as jnp
from jax import lax
from jax.experimental import pallas as pl
from jax.experimental.pallas import tpu as pltpu
from jax.experimental.pallas import tpu_sc as plsc

_SPARSITY = 0.3
_NC = 2
_NS = 16
_L = 16


def _select_level(h, cbuf, nb, r, lane, zeros16):
    nch = nb // _L
    cbuf[pl.ds(nb, _L)] = zeros16

    def chunk_bd(t, acc):
        c = nch - 1 - t
        s = zeros16
        for sl in range(_L):
            gidx = c * (_L * _L) + (lane << 4) + (lane ^ sl)
            s = s + plsc.load_gather(h, [gidx])
        suf = lax.rev(plsc.cumsum(lax.rev(s, (0,))), (0,))
        cbuf[pl.ds(c * _L, _L)] = suf + acc
        return acc + jnp.sum(s)

    lax.fori_loop(0, nch, chunk_bd, jnp.int32(0), unroll=False)

    def cnt_bd(c, accv):
        v = cbuf[pl.ds(c * _L, _L)]
        return accv + jnp.where(v >= r, 1, 0).astype(jnp.int32)

    accv = lax.fori_loop(0, nch, cnt_bd, zeros16, unroll=False)
    B = jnp.sum(accv) - 1
    c_next = cbuf[pl.ds(B + 1, _L)][0]
    return B, r - c_next


def _body(adj_hbm, out_hbm, xbuf, kbuf, h1, h2, h3, h4, cbuf, *, k, rows_per_w, nv):
    wid = lax.axis_index("s") * _NC + lax.axis_index("c")
    lane = lax.iota(jnp.int32, _L)
    zeros16 = jnp.zeros((_L,), jnp.int32)
    ones16 = jnp.ones((_L,), jnp.int32)
    int_min = jnp.int32(-(2 ** 31))

    def clear(h, nwords):
        def bd(i, _):
            h[pl.ds(i * _L, _L)] = zeros16
            return 0
        lax.fori_loop(0, nwords // _L, bd, 0, unroll=False)

    def row_body(j, _):
        row = wid * rows_per_w + j
        pltpu.sync_copy(adj_hbm.at[row], xbuf)
        clear(h1, 512 * _L)
        clear(h2, 256 * _L)
        clear(h3, 256 * _L)
        clear(h4, 128 * _L)

        def sweep_a(i, _):
            x = xbuf[pl.ds(i * _L, _L)]
            b = lax.bitcast_convert_type(x, jnp.int32)
            w = b ^ ((b >> 31) | int_min)
            kbuf[pl.ds(i * _L, _L)] = w
            d = (w >> 23) & 0x1FF
            idx = (d << 4) + ((lane ^ d) & 15)
            plsc.addupdate_scatter(h1, [idx], ones16)
            return 0

        lax.fori_loop(0, nv, sweep_a, 0, unroll=False)
        B1, r1 = _select_level(h1, cbuf, 512, jnp.int32(k), lane, zeros16)

        def sweep_b(i, _):
            w = kbuf[pl.ds(i * _L, _L)]
            cond = ((w >> 23) & 0x1FF) == B1
            d = (w >> 15) & 0xFF
            idx = (d << 4) + ((lane ^ d) & 15)
            plsc.addupdate_scatter(h2, [idx], ones16, mask=cond)
            return 0

        lax.fori_loop(0, nv, sweep_b, 0, unroll=False)
        B2, r2 = _select_level(h2, cbuf, 256, r1, lane, zeros16)
        P2 = (B1 << 8) | B2

        def sweep_c(i, _):
            w = kbuf[pl.ds(i * _L, _L)]
            cond = ((w >> 15) & 0x1FFFF) == P2
            d = (w >> 7) & 0xFF
            idx = (d << 4) + ((lane ^ d) & 15)
            plsc.addupdate_scatter(h3, [idx], ones16, mask=cond)
            return 0

        lax.fori_loop(0, nv, sweep_c, 0, unroll=False)
        B3, r3 = _select_level(h3, cbuf, 256, r2, lane, zeros16)
        P3 = (P2 << 8) | B3

        def sweep_d(i, _):
            w = kbuf[pl.ds(i * _L, _L)]
            cond = ((w >> 7) & 0x1FFFFFF) == P3
            d = w & 0x7F
            idx = (d << 4) + ((lane ^ d) & 15)
            plsc.addupdate_scatter(h4, [idx], ones16, mask=cond)
            return 0

        lax.fori_loop(0, nv, sweep_d, 0, unroll=False)
        B4, _r4 = _select_level(h4, cbuf, 128, r3, lane, zeros16)

        w_thresh = (P3 << 7) | B4
        s_thresh = w_thresh ^ int_min

        def sweep_e(i, _):
            w = kbuf[pl.ds(i * _L, _L)]
            s = w ^ int_min
            x = xbuf[pl.ds(i * _L, _L)]
            xbuf[pl.ds(i * _L, _L)] = jnp.where(s >= s_thresh, x, 0.0)
            return 0

        lax.fori_loop(0, nv, sweep_e, 0, unroll=False)
        pltpu.sync_copy(xbuf, out_hbm.at[row])
        return 0

    lax.fori_loop(0, rows_per_w, row_body, 0, unroll=False)


def kernel(adj):
    rows, n = adj.shape
    k = max(1, int(n * (1.0 - _SPARSITY)))
    nw = _NC * _NS
    assert rows % nw == 0 and n % _L == 0
    rows_per_w = rows // nw
    nv = n // _L

    mesh = plsc.VectorSubcoreMesh(
        core_axis_name="c", subcore_axis_name="s", num_cores=_NC, num_subcores=_NS
    )
    f = pl.kernel(
        functools.partial(_body, k=k, rows_per_w=rows_per_w, nv=nv),
        out_type=jax.ShapeDtypeStruct((rows, n), jnp.float32),
        mesh=mesh,
        compiler_params=pltpu.CompilerParams(needs_layout_passes=False),
        scratch_types=[
            pltpu.VMEM((n,), jnp.float32),
            pltpu.VMEM((n,), jnp.int32),
            pltpu.VMEM((512 * _L,), jnp.int32),
            pltpu.VMEM((256 * _L,), jnp.int32),
            pltpu.VMEM((256 * _L,), jnp.int32),
            pltpu.VMEM((128 * _L,), jnp.int32),
            pltpu.VMEM((512 + 2 * _L,), jnp.int32),
        ],
    )
    return f(adj)


# speedup vs baseline: 128.9469x; 3.5265x over previous
import functools

import jax
import jax.numpy as jnp
from jax import lax
from jax.experimental import pallas as pl
from jax.experimental.pallas import tpu as pltpu
from jax.experimental.pallas import tpu_sc as plsc

_SPARSITY = 0.3
_NC = 2
_NS = 16
_L = 16


def _select_level(h, cbuf, nb, r, lane, zeros16):
    nch = nb // _L
    cbuf[pl.ds(nb, _L)] = zeros16

    def chunk_bd(t, acc):
        c = nch - 1 - t
        s = zeros16
        for sl in range(_L):
            gidx = c * (_L * _L) + (lane << 4) + (lane ^ sl)
            s = s + plsc.load_gather(h, [gidx])
        suf = lax.rev(plsc.cumsum(lax.rev(s, (0,))), (0,))
        cbuf[pl.ds(c * _L, _L)] = suf + acc
        return acc + jnp.sum(s)

    lax.fori_loop(0, nch, chunk_bd, jnp.int32(0), unroll=False)

    def cnt_bd(c, accv):
        v = cbuf[pl.ds(c * _L, _L)]
        return accv + jnp.where(v >= r, 1, 0).astype(jnp.int32)

    accv = lax.fori_loop(0, nch, cnt_bd, zeros16, unroll=False)
    B = jnp.sum(accv) - 1
    c_next = cbuf[pl.ds(B + 1, _L)][0]
    return B, r - c_next


def _body(adj_hbm, out_hbm, xbuf0, xbuf1, kbuf, h1, h2, h3, h4, cbuf,
          sem_i0, sem_i1, sem_o0, sem_o1, *, k, rows_per_w, nv):
    wid = lax.axis_index("s") * _NC + lax.axis_index("c")
    lane = lax.iota(jnp.int32, _L)
    zeros16 = jnp.zeros((_L,), jnp.int32)
    ones16 = jnp.ones((_L,), jnp.int32)
    int_min = jnp.int32(-(2 ** 31))

    def clear(h, nwords):
        @plsc.parallel_loop(0, nwords // _L, unroll=4)
        def _(i):
            h[pl.ds(i * _L, _L)] = zeros16

    def row_compute(xbuf, row_idx):
        clear(h1, 512 * _L)
        clear(h2, 256 * _L)
        clear(h3, 256 * _L)
        clear(h4, 128 * _L)

        @plsc.parallel_loop(0, nv, unroll=4)
        def sweep_a(i):
            x = xbuf[pl.ds(i * _L, _L)]
            b = lax.bitcast_convert_type(x, jnp.int32)
            w = b ^ ((b >> 31) | int_min)
            kbuf[pl.ds(i * _L, _L)] = w
            d = (w >> 23) & 0x1FF
            idx = (d << 4) + ((lane ^ d) & 15)
            plsc.addupdate_scatter(h1, [idx], ones16)
        B1, r1 = _select_level(h1, cbuf, 512, jnp.int32(k), lane, zeros16)

        @plsc.parallel_loop(0, nv, unroll=4)
        def sweep_b(i):
            w = kbuf[pl.ds(i * _L, _L)]
            cond = ((w >> 23) & 0x1FF) == B1
            d = (w >> 15) & 0xFF
            idx = (d << 4) + ((lane ^ d) & 15)
            plsc.addupdate_scatter(h2, [idx], ones16, mask=cond)

        B2, r2 = _select_level(h2, cbuf, 256, r1, lane, zeros16)
        P2 = (B1 << 8) | B2

        @plsc.parallel_loop(0, nv, unroll=4)
        def sweep_c(i):
            w = kbuf[pl.ds(i * _L, _L)]
            cond = ((w >> 15) & 0x1FFFF) == P2
            d = (w >> 7) & 0xFF
            idx = (d << 4) + ((lane ^ d) & 15)
            plsc.addupdate_scatter(h3, [idx], ones16, mask=cond)

        B3, r3 = _select_level(h3, cbuf, 256, r2, lane, zeros16)
        P3 = (P2 << 8) | B3

        @plsc.parallel_loop(0, nv, unroll=4)
        def sweep_d(i):
            w = kbuf[pl.ds(i * _L, _L)]
            cond = ((w >> 7) & 0x1FFFFFF) == P3
            d = w & 0x7F
            idx = (d << 4) + ((lane ^ d) & 15)
            plsc.addupdate_scatter(h4, [idx], ones16, mask=cond)

        B4, _r4 = _select_level(h4, cbuf, 128, r3, lane, zeros16)

        w_thresh = (P3 << 7) | B4
        s_thresh = w_thresh ^ int_min

        @plsc.parallel_loop(0, nv, unroll=4)
        def sweep_e(i):
            w = kbuf[pl.ds(i * _L, _L)]
            s = w ^ int_min
            x = xbuf[pl.ds(i * _L, _L)]
            xbuf[pl.ds(i * _L, _L)] = jnp.where(s >= s_thresh, x, 0.0)


    rows = [wid * rows_per_w + r for r in range(rows_per_w)]
    bufs = [xbuf0, xbuf1]
    sem_in = [sem_i0, sem_i1]
    sem_out = [sem_o0, sem_o1]
    ins = {0: pltpu.async_copy(adj_hbm.at[rows[0]], bufs[0], sem_in[0])}
    outs = {}
    for r in range(rows_per_w):
        b = r % 2
        ins[r].wait()
        if r + 1 < rows_per_w:
            if r >= 1:
                outs[r - 1].wait()
            ins[r + 1] = pltpu.async_copy(
                adj_hbm.at[rows[r + 1]], bufs[(r + 1) % 2], sem_in[(r + 1) % 2]
            )
        row_compute(bufs[b], rows[r])
        outs[r] = pltpu.async_copy(bufs[b], out_hbm.at[rows[r]], sem_out[b])
    if rows_per_w >= 2:
        outs[rows_per_w - 2].wait()
    outs[rows_per_w - 1].wait()


def kernel(adj):
    rows, n = adj.shape
    k = max(1, int(n * (1.0 - _SPARSITY)))
    nw = _NC * _NS
    assert rows % nw == 0 and n % _L == 0
    rows_per_w = rows // nw
    nv = n // _L

    mesh = plsc.VectorSubcoreMesh(
        core_axis_name="c", subcore_axis_name="s", num_cores=_NC, num_subcores=_NS
    )
    f = pl.kernel(
        functools.partial(_body, k=k, rows_per_w=rows_per_w, nv=nv),
        out_type=jax.ShapeDtypeStruct((rows, n), jnp.float32),
        mesh=mesh,
        compiler_params=pltpu.CompilerParams(needs_layout_passes=False),
        scratch_types=[
            pltpu.VMEM((n,), jnp.float32),
            pltpu.VMEM((n,), jnp.float32),
            pltpu.VMEM((n,), jnp.int32),
            pltpu.VMEM((512 * _L,), jnp.int32),
            pltpu.VMEM((256 * _L,), jnp.int32),
            pltpu.VMEM((256 * _L,), jnp.int32),
            pltpu.VMEM((128 * _L,), jnp.int32),
            pltpu.VMEM((512 + 2 * _L,), jnp.int32),
            pltpu.SemaphoreType.DMA,
            pltpu.SemaphoreType.DMA,
            pltpu.SemaphoreType.DMA,
            pltpu.SemaphoreType.DMA,
        ],
    )
    return f(adj)
